# baseline (speedup 1.0000x reference)
import functools

import jax
import jax.numpy as jnp
from jax import lax
from jax.experimental import pallas as pl
from jax.experimental.pallas import tpu as pltpu
from jax.experimental.pallas import tpu_sc as plsc

_K = 8192
_D = 32
_EPS = 1e-5
_TOK_BLK = 128
_K_BLK = 512
_K_STEPS = _K // _K_BLK
_N_TOK = 6272
_N_BLKS = _N_TOK // _TOK_BLK
_PAD_TOK = 8192
_GLANES = 128
_LOSS_SCALE = 1.25 / (_N_TOK * _D)


def _vq_tc_body(q_ref, e_ref, qq_ref, ee_ref, closest_ref, bv_ref, bi_ref):
    q = q_ref[...]
    qq = qq_ref[...]
    q_bf = q.astype(jnp.bfloat16)

    bv_ref[...] = jnp.full((_TOK_BLK, 1), jnp.inf, jnp.float32)
    bi_ref[...] = jnp.zeros((_TOK_BLK, 1), jnp.int32)

    def body(j, carry):
        off = pl.multiple_of(j * _K_BLK, _K_BLK)
        e_blk = e_ref[pl.ds(off, _K_BLK), :]
        ee = ee_ref[j, :][None, :]
        dot = lax.dot_general(q_bf, e_blk.astype(jnp.bfloat16),
                              (((1,), (1,)), ((), ())),
                              preferred_element_type=jnp.float32)
        d2 = (qq + ee) - 2.0 * dot
        dist = jnp.sqrt(jnp.maximum(d2, 0.0))
        local_val = jnp.min(dist, axis=1, keepdims=True)
        kiota = lax.broadcasted_iota(jnp.int32, (_TOK_BLK, _K_BLK), 1)
        cand = jnp.where(dist == local_val, kiota, _K_BLK)
        local_idx = jnp.min(cand, axis=1, keepdims=True) + j * _K_BLK
        bv = bv_ref[...]
        better = local_val < bv
        bv_ref[...] = jnp.where(better, local_val, bv)
        bi_ref[...] = jnp.where(better, local_idx, bi_ref[...])
        return carry

    lax.fori_loop(0, _K_STEPS, body, 0)
    closest_ref[0, :, :] = bi_ref[...]


def _vq_argmin(q2d, emb, qq, ee):
    closest3d = pl.pallas_call(
        _vq_tc_body,
        grid=(_N_BLKS,),
        in_specs=[
            pl.BlockSpec((_TOK_BLK, _D), lambda i: (i, 0)),
            pl.BlockSpec((_K, _D), lambda i: (0, 0)),
            pl.BlockSpec((_TOK_BLK, 1), lambda i: (i, 0)),
            pl.BlockSpec((_K_STEPS, _K_BLK), lambda i: (0, 0)),
        ],
        out_specs=pl.BlockSpec((1, _TOK_BLK, 1), lambda i: (i, 0, 0)),
        out_shape=jax.ShapeDtypeStruct((_N_BLKS, _TOK_BLK, 1), jnp.int32),
        scratch_shapes=[
            pltpu.VMEM((_TOK_BLK, 1), jnp.float32),
            pltpu.VMEM((_TOK_BLK, 1), jnp.int32),
        ],
    )(q2d, emb, qq, ee)
    return closest3d.reshape(-1)


def _mse_body(a_ref, b_ref, o_ref):
    diff = a_ref[...] - b_ref[...]
    o_ref[...] = (jnp.sum(diff * diff) * _LOSS_SCALE).reshape(1, 1)


def _vq_loss(quant, q2d):
    loss2d = pl.pallas_call(
        _mse_body,
        out_shape=jax.ShapeDtypeStruct((1, 1), jnp.float32),
    )(quant, q2d)
    return loss2d[0, 0]


def _sc_gather(table128, idx_padded):
    info = plsc.get_sparse_core_info()
    nw = info.num_cores * info.num_subcores
    bpw = _PAD_TOK // nw
    nch = bpw // _GLANES
    mesh = plsc.VectorSubcoreMesh(core_axis_name="c", subcore_axis_name="s")

    @functools.partial(
        pl.kernel, mesh=mesh,
        out_type=jax.ShapeDtypeStruct((_PAD_TOK, _GLANES), jnp.float32),
        scratch_types=[
            pltpu.VMEM((nch, _GLANES), jnp.int32),
            pltpu.VMEM((nch, _GLANES, _GLANES), jnp.float32),
            pltpu.SemaphoreType.DMA,
        ],
    )
    def gather(table_hbm, idx_hbm, out_hbm, idx_v, rows_v, sem):
        wid = lax.axis_index("s") * info.num_cores + lax.axis_index("c")
        base = wid * bpw
        pltpu.sync_copy(
            idx_hbm.at[pl.ds(wid * nch, nch)], idx_v)
        for j in range(nch):
            pltpu.async_copy(table_hbm.at[idx_v.at[j]], rows_v.at[j], sem).wait()
            pltpu.sync_copy(
                rows_v.at[j], out_hbm.at[pl.ds(base + j * _GLANES, _GLANES)])

    return gather(table128, idx_padded.reshape(nw * nch, _GLANES))


def _conv(x, w, b, stride, pad):
    out = lax.conv_general_dilated(
        x, w, (stride, stride), ((pad, pad), (pad, pad)),
        dimension_numbers=('NCHW', 'OIHW', 'NCHW'))
    return out + b[None, :, None, None]


def _convT(x, w, b, stride, pad):
    k = w.shape[2]
    w2 = jnp.transpose(jnp.flip(w, axis=(2, 3)), (1, 0, 2, 3))
    out = lax.conv_general_dilated(
        x, w2, (1, 1), ((k - 1 - pad, k - 1 - pad), (k - 1 - pad, k - 1 - pad)),
        lhs_dilation=(stride, stride), dimension_numbers=('NCHW', 'OIHW', 'NCHW'))
    return out + b[None, :, None, None]


def _bn(x, g, b):
    m = jnp.mean(x, axis=(0, 2, 3), keepdims=True)
    v = jnp.var(x, axis=(0, 2, 3), keepdims=True)
    return (x - m) / jnp.sqrt(v + _EPS) * g[None, :, None, None] + b[None, :, None, None]


def _relu(x):
    return jnp.maximum(x, 0.0)


def _encoder(x, p):
    h = _relu(_bn(_conv(x, p['enc_c1_w'], p['enc_c1_b'], 2, 1), p['enc_bn1_g'], p['enc_bn1_b']))
    h = _relu(_bn(_conv(h, p['enc_c2_w'], p['enc_c2_b'], 2, 1), p['enc_bn2_g'], p['enc_bn2_b']))
    h = h + _relu(_bn(_conv(h, p['enc_r1_w'], p['enc_r1_b'], 1, 1), p['enc_r1bn_g'], p['enc_r1bn_b']))
    h = h + _relu(_bn(_conv(h, p['enc_r2_w'], p['enc_r2_b'], 1, 0), p['enc_r2bn_g'], p['enc_r2bn_b']))
    return _conv(h, p['enc_proj_w'], p['enc_proj_b'], 1, 0)


def _decoder(z, p):
    h = _conv(z, p['dec_proj_w'], p['dec_proj_b'], 1, 0)
    h = h + _relu(_bn(_conv(h, p['dec_r1_w'], p['dec_r1_b'], 1, 1), p['dec_r1bn_g'], p['dec_r1bn_b']))
    h = h + _relu(_bn(_conv(h, p['dec_r2_w'], p['dec_r2_b'], 1, 1), p['dec_r2bn_g'], p['dec_r2bn_b']))
    h = _relu(_bn(_convT(h, p['dec_ct1_w'], p['dec_ct1_b'], 2, 1), p['dec_ctbn_g'], p['dec_ctbn_b']))
    h = _convT(h, p['dec_ct2_w'], p['dec_ct2_b'], 2, 1)
    return jax.nn.sigmoid(h)


def kernel(x, params):
    enc = _encoder(x, params)
    b, c, hh, ww = enc.shape
    q2d = enc.reshape(-1, _D)
    emb = params['emb']
    qq = jnp.sum(enc.reshape(b, -1, c) * enc.reshape(b, -1, c), -1)
    ee = jnp.sum(emb * emb, -1)
    closest_flat = _vq_argmin(q2d, emb, qq.reshape(_N_TOK, 1),
                              ee.reshape(_K_STEPS, _K_BLK))
    idx_padded = jnp.concatenate(
        [closest_flat, jnp.zeros((_PAD_TOK - _N_TOK,), jnp.int32)])
    table128 = jnp.pad(emb, ((0, 0), (0, _GLANES - _D)))
    quant = _sc_gather(table128, idx_padded)[:_N_TOK, :_D]
    qloss = _vq_loss(quant, q2d)
    quant_out = (q2d + (quant - q2d)).reshape(b, c, hh, ww)
    out = _decoder(quant_out, params)
    closest = closest_flat.reshape(b, -1)
    return out, closest, qloss


# speedup vs baseline: 1.4446x; 1.4446x over previous
import functools

import jax
import jax.numpy as jnp
from jax import lax
from jax.experimental import pallas as pl
from jax.experimental.pallas import tpu as pltpu
from jax.experimental.pallas import tpu_sc as plsc

_K = 8192
_D = 32
_EPS = 1e-5
_TOK_BLK = 128
_K_BLK = 512
_K_STEPS = _K // _K_BLK
_N_TOK = 6272
_N_BLKS = _N_TOK // _TOK_BLK
_PAD_TOK = 8192
_GLANES = 128
_LOSS_SCALE = 1.25 / (_N_TOK * _D)


def _vq_tc_body(q_ref, e_ref, he_ref, closest_ref, loss_ref):
    i = pl.program_id(0)
    q = q_ref[...]
    q_bf = q.astype(jnp.bfloat16)
    kiota = lax.broadcasted_iota(jnp.int32, (_TOK_BLK, _K_BLK), 1)

    bv = jnp.full((_TOK_BLK, 1), -jnp.inf, jnp.float32)
    bi = jnp.zeros((_TOK_BLK, 1), jnp.int32)
    for j in range(_K_STEPS):
        dot = lax.dot_general(q_bf, e_ref[pl.ds(j * _K_BLK, _K_BLK), :],
                              (((1,), (1,)), ((), ())),
                              preferred_element_type=jnp.float32)
        s = dot - he_ref[j, :][None, :]
        m = jnp.max(s, axis=1, keepdims=True)
        cand = jnp.where(s == m, kiota, _K_BLK)
        li = jnp.min(cand, axis=1, keepdims=True) + j * _K_BLK
        better = m > bv
        bi = jnp.where(better, li, bi)
        bv = jnp.where(better, m, bv)
    closest_ref[0, :, :] = bi

    blk_loss = jnp.sum(q * q) - 2.0 * jnp.sum(bv)
    prev = jnp.where(i == 0, jnp.zeros((1, 1), jnp.float32), loss_ref[...])
    tot = prev + blk_loss
    loss_ref[...] = jnp.where(i == _N_BLKS - 1, tot * _LOSS_SCALE, tot)


def _vq_argmin_loss(q2d, emb_bf, he):
    closest3d, loss2d = pl.pallas_call(
        _vq_tc_body,
        grid=(_N_BLKS,),
        in_specs=[
            pl.BlockSpec((_TOK_BLK, _D), lambda i: (i, 0)),
            pl.BlockSpec((_K, _D), lambda i: (0, 0)),
            pl.BlockSpec((_K_STEPS, _K_BLK), lambda i: (0, 0)),
        ],
        out_specs=[
            pl.BlockSpec((1, _TOK_BLK, 1), lambda i: (i, 0, 0)),
            pl.BlockSpec((1, 1), lambda i: (0, 0)),
        ],
        out_shape=[
            jax.ShapeDtypeStruct((_N_BLKS, _TOK_BLK, 1), jnp.int32),
            jax.ShapeDtypeStruct((1, 1), jnp.float32),
        ],
    )(q2d, emb_bf, he)
    return closest3d.reshape(-1), loss2d[0, 0]


def _sc_gather(table128, idx_padded):
    info = plsc.get_sparse_core_info()
    nw = info.num_cores * info.num_subcores
    bpw = _PAD_TOK // nw
    nch = bpw // _GLANES
    mesh = plsc.VectorSubcoreMesh(core_axis_name="c", subcore_axis_name="s")

    @functools.partial(
        pl.kernel, mesh=mesh,
        out_type=jax.ShapeDtypeStruct((_PAD_TOK, _GLANES), jnp.float32),
        scratch_types=[
            pltpu.VMEM((nch, _GLANES), jnp.int32),
            pltpu.VMEM((nch, _GLANES, _GLANES), jnp.float32),
            pltpu.SemaphoreType.DMA,
        ],
    )
    def gather(table_hbm, idx_hbm, out_hbm, idx_v, rows_v, sem):
        wid = lax.axis_index("s") * info.num_cores + lax.axis_index("c")
        base = wid * bpw
        pltpu.sync_copy(
            idx_hbm.at[pl.ds(wid * nch, nch)], idx_v)
        copies = [pltpu.async_copy(table_hbm.at[idx_v.at[j]], rows_v.at[j], sem)
                  for j in range(nch)]
        for j in range(nch):
            copies[j].wait()
            pltpu.sync_copy(
                rows_v.at[j], out_hbm.at[pl.ds(base + j * _GLANES, _GLANES)])

    return gather(table128, idx_padded.reshape(nw * nch, _GLANES))


def _conv(x, w, b, stride, pad):
    out = lax.conv_general_dilated(
        x, w, (stride, stride), ((pad, pad), (pad, pad)),
        dimension_numbers=('NCHW', 'OIHW', 'NCHW'))
    return out + b[None, :, None, None]


def _convT(x, w, b, stride, pad):
    k = w.shape[2]
    w2 = jnp.transpose(jnp.flip(w, axis=(2, 3)), (1, 0, 2, 3))
    out = lax.conv_general_dilated(
        x, w2, (1, 1), ((k - 1 - pad, k - 1 - pad), (k - 1 - pad, k - 1 - pad)),
        lhs_dilation=(stride, stride), dimension_numbers=('NCHW', 'OIHW', 'NCHW'))
    return out + b[None, :, None, None]


def _bn(x, g, b):
    m = jnp.mean(x, axis=(0, 2, 3), keepdims=True)
    v = jnp.var(x, axis=(0, 2, 3), keepdims=True)
    return (x - m) / jnp.sqrt(v + _EPS) * g[None, :, None, None] + b[None, :, None, None]


def _relu(x):
    return jnp.maximum(x, 0.0)


def _encoder(x, p):
    h = _relu(_bn(_conv(x, p['enc_c1_w'], p['enc_c1_b'], 2, 1), p['enc_bn1_g'], p['enc_bn1_b']))
    h = _relu(_bn(_conv(h, p['enc_c2_w'], p['enc_c2_b'], 2, 1), p['enc_bn2_g'], p['enc_bn2_b']))
    h = h + _relu(_bn(_conv(h, p['enc_r1_w'], p['enc_r1_b'], 1, 1), p['enc_r1bn_g'], p['enc_r1bn_b']))
    h = h + _relu(_bn(_conv(h, p['enc_r2_w'], p['enc_r2_b'], 1, 0), p['enc_r2bn_g'], p['enc_r2bn_b']))
    return _conv(h, p['enc_proj_w'], p['enc_proj_b'], 1, 0)


def _decoder(z, p):
    h = _conv(z, p['dec_proj_w'], p['dec_proj_b'], 1, 0)
    h = h + _relu(_bn(_conv(h, p['dec_r1_w'], p['dec_r1_b'], 1, 1), p['dec_r1bn_g'], p['dec_r1bn_b']))
    h = h + _relu(_bn(_conv(h, p['dec_r2_w'], p['dec_r2_b'], 1, 1), p['dec_r2bn_g'], p['dec_r2bn_b']))
    h = _relu(_bn(_convT(h, p['dec_ct1_w'], p['dec_ct1_b'], 2, 1), p['dec_ctbn_g'], p['dec_ctbn_b']))
    h = _convT(h, p['dec_ct2_w'], p['dec_ct2_b'], 2, 1)
    return jax.nn.sigmoid(h)


def kernel(x, params):
    enc = _encoder(x, params)
    b, c, hh, ww = enc.shape
    q2d = enc.reshape(-1, _D)
    emb = params['emb']
    emb_bf = emb.astype(jnp.bfloat16)
    he = (0.5 * jnp.sum(emb * emb, -1)).reshape(_K_STEPS, _K_BLK)
    closest_flat, qloss = _vq_argmin_loss(q2d, emb_bf, he)
    idx_padded = jnp.concatenate(
        [closest_flat, jnp.zeros((_PAD_TOK - _N_TOK,), jnp.int32)])
    table128 = jnp.pad(emb, ((0, 0), (0, _GLANES - _D)))
    quant = _sc_gather(table128, idx_padded)[:_N_TOK, :_D]
    quant_out = (q2d + (quant - q2d)).reshape(b, c, hh, ww)
    out = _decoder(quant_out, params)
    closest = closest_flat.reshape(b, -1)
    return out, closest, qloss


# speedup vs baseline: 1.5654x; 1.0836x over previous
import functools

import jax
import jax.numpy as jnp
from jax import lax
from jax.experimental import pallas as pl
from jax.experimental.pallas import tpu as pltpu
from jax.experimental.pallas import tpu_sc as plsc

_K = 8192
_D = 32
_EPS = 1e-5
_TOK_BLK = 128
_K_BLK = 512
_K_STEPS = _K // _K_BLK
_N_TOK = 6272
_N_BLKS = _N_TOK // _TOK_BLK
_PAD_TOK = 8192
_GLANES = 128
_LOSS_SCALE = 1.25 / (_N_TOK * _D)


def _vq_tc_body(q_ref, e_ref, he_ref, closest_ref, loss_ref):
    i = pl.program_id(0)
    q = q_ref[...]
    q_bf = q.astype(jnp.bfloat16)
    kiota = lax.broadcasted_iota(jnp.int32, (_TOK_BLK, _K_BLK), 1)

    bv = jnp.full((_TOK_BLK, 1), -jnp.inf, jnp.float32)
    bi = jnp.zeros((_TOK_BLK, 1), jnp.int32)
    for j in range(_K_STEPS):
        dot = lax.dot_general(q_bf, e_ref[pl.ds(j * _K_BLK, _K_BLK), :],
                              (((1,), (1,)), ((), ())),
                              preferred_element_type=jnp.float32)
        s = dot - he_ref[j, :][None, :]
        m = jnp.max(s, axis=1, keepdims=True)
        cand = jnp.where(s == m, kiota, _K_BLK)
        li = jnp.min(cand, axis=1, keepdims=True) + j * _K_BLK
        better = m > bv
        bi = jnp.where(better, li, bi)
        bv = jnp.where(better, m, bv)
    closest_ref[0, :, :] = bi

    blk_loss = jnp.sum(q * q) - 2.0 * jnp.sum(bv)
    prev = jnp.where(i == 0, jnp.zeros((1, 1), jnp.float32), loss_ref[...])
    tot = prev + blk_loss
    loss_ref[...] = jnp.where(i == _N_BLKS - 1, tot * _LOSS_SCALE, tot)


def _vq_argmin_loss(q2d, emb_bf, he):
    closest3d, loss2d = pl.pallas_call(
        _vq_tc_body,
        grid=(_N_BLKS,),
        in_specs=[
            pl.BlockSpec((_TOK_BLK, _D), lambda i: (i, 0)),
            pl.BlockSpec((_K, _D), lambda i: (0, 0)),
            pl.BlockSpec((_K_STEPS, _K_BLK), lambda i: (0, 0)),
        ],
        out_specs=[
            pl.BlockSpec((1, _TOK_BLK, 1), lambda i: (i, 0, 0)),
            pl.BlockSpec((1, 1), lambda i: (0, 0)),
        ],
        out_shape=[
            jax.ShapeDtypeStruct((_N_BLKS, _TOK_BLK, 1), jnp.int32),
            jax.ShapeDtypeStruct((1, 1), jnp.float32),
        ],
    )(q2d, emb_bf, he)
    return closest3d.reshape(-1), loss2d[0, 0]


def _sc_gather(table128, idx_padded):
    info = plsc.get_sparse_core_info()
    nw = info.num_cores * info.num_subcores
    bpw = _PAD_TOK // nw
    nch = bpw // _GLANES
    mesh = plsc.VectorSubcoreMesh(core_axis_name="c", subcore_axis_name="s")

    @functools.partial(
        pl.kernel, mesh=mesh,
        out_type=jax.ShapeDtypeStruct((_PAD_TOK, _D), jnp.float32),
        scratch_types=[
            pltpu.VMEM((nch, _GLANES), jnp.int32),
            pltpu.VMEM((nch, _GLANES, _D), jnp.float32),
            pltpu.SemaphoreType.DMA,
        ],
        compiler_params=pltpu.CompilerParams(use_tc_tiling_on_sc=False),
    )
    def gather(table_hbm, idx_hbm, out_hbm, idx_v, rows_v, sem):
        wid = lax.axis_index("s") * info.num_cores + lax.axis_index("c")
        base = wid * bpw
        pltpu.sync_copy(
            idx_hbm.at[pl.ds(wid * nch, nch)], idx_v)
        copies = [pltpu.async_copy(table_hbm.at[idx_v.at[j]], rows_v.at[j], sem)
                  for j in range(nch)]
        for j in range(nch):
            copies[j].wait()
            pltpu.sync_copy(
                rows_v.at[j], out_hbm.at[pl.ds(base + j * _GLANES, _GLANES)])

    return gather(table128, idx_padded.reshape(nw * nch, _GLANES))


def _conv(x, w, b, stride, pad):
    out = lax.conv_general_dilated(
        x, w, (stride, stride), ((pad, pad), (pad, pad)),
        dimension_numbers=('NCHW', 'OIHW', 'NCHW'))
    return out + b[None, :, None, None]


def _convT(x, w, b, stride, pad):
    k = w.shape[2]
    w2 = jnp.transpose(jnp.flip(w, axis=(2, 3)), (1, 0, 2, 3))
    out = lax.conv_general_dilated(
        x, w2, (1, 1), ((k - 1 - pad, k - 1 - pad), (k - 1 - pad, k - 1 - pad)),
        lhs_dilation=(stride, stride), dimension_numbers=('NCHW', 'OIHW', 'NCHW'))
    return out + b[None, :, None, None]


def _bn(x, g, b):
    m = jnp.mean(x, axis=(0, 2, 3), keepdims=True)
    v = jnp.var(x, axis=(0, 2, 3), keepdims=True)
    return (x - m) / jnp.sqrt(v + _EPS) * g[None, :, None, None] + b[None, :, None, None]


def _relu(x):
    return jnp.maximum(x, 0.0)


def _encoder(x, p):
    h = _relu(_bn(_conv(x, p['enc_c1_w'], p['enc_c1_b'], 2, 1), p['enc_bn1_g'], p['enc_bn1_b']))
    h = _relu(_bn(_conv(h, p['enc_c2_w'], p['enc_c2_b'], 2, 1), p['enc_bn2_g'], p['enc_bn2_b']))
    h = h + _relu(_bn(_conv(h, p['enc_r1_w'], p['enc_r1_b'], 1, 1), p['enc_r1bn_g'], p['enc_r1bn_b']))
    h = h + _relu(_bn(_conv(h, p['enc_r2_w'], p['enc_r2_b'], 1, 0), p['enc_r2bn_g'], p['enc_r2bn_b']))
    return _conv(h, p['enc_proj_w'], p['enc_proj_b'], 1, 0)


def _decoder(z, p):
    h = _conv(z, p['dec_proj_w'], p['dec_proj_b'], 1, 0)
    h = h + _relu(_bn(_conv(h, p['dec_r1_w'], p['dec_r1_b'], 1, 1), p['dec_r1bn_g'], p['dec_r1bn_b']))
    h = h + _relu(_bn(_conv(h, p['dec_r2_w'], p['dec_r2_b'], 1, 1), p['dec_r2bn_g'], p['dec_r2bn_b']))
    h = _relu(_bn(_convT(h, p['dec_ct1_w'], p['dec_ct1_b'], 2, 1), p['dec_ctbn_g'], p['dec_ctbn_b']))
    h = _convT(h, p['dec_ct2_w'], p['dec_ct2_b'], 2, 1)
    return jax.nn.sigmoid(h)


def kernel(x, params):
    enc = _encoder(x, params)
    b, c, hh, ww = enc.shape
    q2d = enc.reshape(-1, _D)
    emb = params['emb']
    emb_bf = emb.astype(jnp.bfloat16)
    he = (0.5 * jnp.sum(emb * emb, -1)).reshape(_K_STEPS, _K_BLK)
    closest_flat, qloss = _vq_argmin_loss(q2d, emb_bf, he)
    idx_padded = jnp.concatenate(
        [closest_flat, jnp.zeros((_PAD_TOK - _N_TOK,), jnp.int32)])
    quant = _sc_gather(emb, idx_padded)[:_N_TOK]
    quant_out = (q2d + (quant - q2d)).reshape(b, c, hh, ww)
    out = _decoder(quant_out, params)
    closest = closest_flat.reshape(b, -1)
    return out, closest, qloss


# speedup vs baseline: 1.5659x; 1.0003x over previous
import functools

import jax
import jax.numpy as jnp
from jax import lax
from jax.experimental import pallas as pl
from jax.experimental.pallas import tpu as pltpu
from jax.experimental.pallas import tpu_sc as plsc

_K = 8192
_D = 32
_EPS = 1e-5
_TOK_BLK = 128
_K_BLK = 512
_K_STEPS = _K // _K_BLK
_N_TOK = 6272
_N_BLKS = _N_TOK // _TOK_BLK
_PAD_TOK = 8192
_GLANES = 128
_LOSS_SCALE = 1.25 / (_N_TOK * _D)


def _vq_tc_body(q_ref, e_ref, he_ref, closest_ref, loss_ref):
    i = pl.program_id(0)
    q = q_ref[...]
    q_bf = q.astype(jnp.bfloat16)
    kiota = lax.broadcasted_iota(jnp.int32, (_TOK_BLK, _K_BLK), 1)

    bv = jnp.full((_TOK_BLK, 1), -jnp.inf, jnp.float32)
    bi = jnp.zeros((_TOK_BLK, 1), jnp.int32)
    for j in range(_K_STEPS):
        dot = lax.dot_general(q_bf, e_ref[pl.ds(j * _K_BLK, _K_BLK), :],
                              (((1,), (1,)), ((), ())),
                              preferred_element_type=jnp.float32)
        s = dot - he_ref[j, :][None, :]
        m = jnp.max(s, axis=1, keepdims=True)
        cand = jnp.where(s == m, kiota, _K_BLK)
        li = jnp.min(cand, axis=1, keepdims=True) + j * _K_BLK
        better = m > bv
        bi = jnp.where(better, li, bi)
        bv = jnp.where(better, m, bv)
    closest_ref[0, :, :] = bi

    blk_loss = jnp.sum(q * q) - 2.0 * jnp.sum(bv)
    prev = jnp.where(i == 0, jnp.zeros((1, 1), jnp.float32), loss_ref[...])
    tot = prev + blk_loss
    loss_ref[...] = jnp.where(i == _N_BLKS - 1, tot * _LOSS_SCALE, tot)


def _vq_argmin_loss(q2d, emb_bf, he):
    closest3d, loss2d = pl.pallas_call(
        _vq_tc_body,
        grid=(_N_BLKS,),
        in_specs=[
            pl.BlockSpec((_TOK_BLK, _D), lambda i: (i, 0)),
            pl.BlockSpec((_K, _D), lambda i: (0, 0)),
            pl.BlockSpec((_K_STEPS, _K_BLK), lambda i: (0, 0)),
        ],
        out_specs=[
            pl.BlockSpec((1, _TOK_BLK, 1), lambda i: (i, 0, 0)),
            pl.BlockSpec((1, 1), lambda i: (0, 0)),
        ],
        out_shape=[
            jax.ShapeDtypeStruct((_N_BLKS, _TOK_BLK, 1), jnp.int32),
            jax.ShapeDtypeStruct((1, 1), jnp.float32),
        ],
    )(q2d, emb_bf, he)
    return closest3d.reshape(-1), loss2d[0, 0]


def _sc_gather(table, idx_padded):
    info = plsc.get_sparse_core_info()
    nw = info.num_cores * info.num_subcores
    bpw = _PAD_TOK // nw
    nch = bpw // _GLANES
    mesh = plsc.VectorSubcoreMesh(core_axis_name="c", subcore_axis_name="s")

    @functools.partial(
        pl.kernel, mesh=mesh,
        out_type=jax.ShapeDtypeStruct((_PAD_TOK, _D), jnp.float32),
        scratch_types=[
            pltpu.VMEM((nch, _GLANES), jnp.int32),
            pltpu.VMEM((nch, _GLANES, _D), jnp.float32),
            pltpu.SemaphoreType.DMA,
        ],
        compiler_params=pltpu.CompilerParams(use_tc_tiling_on_sc=False),
    )
    def gather(table_hbm, idx_hbm, out_hbm, idx_v, rows_v, sem):
        wid = lax.axis_index("s") * info.num_cores + lax.axis_index("c")
        base = wid * bpw
        pltpu.sync_copy(
            idx_hbm.at[pl.ds(wid * nch, nch)], idx_v)
        copies = [pltpu.async_copy(table_hbm.at[idx_v.at[j]], rows_v.at[j], sem)
                  for j in range(nch)]
        for j in range(nch):
            copies[j].wait()
            pltpu.sync_copy(
                rows_v.at[j], out_hbm.at[pl.ds(base + j * _GLANES, _GLANES)])

    return gather(table, idx_padded.reshape(nw * nch, _GLANES))


def _conv(x, w, b, stride, pad):
    out = lax.conv_general_dilated(
        x, w, (stride, stride), ((pad, pad), (pad, pad)),
        dimension_numbers=('NCHW', 'OIHW', 'NCHW'))
    return out + b[None, :, None, None]


def _convT(x, w, b, stride, pad):
    k = w.shape[2]
    w2 = jnp.transpose(jnp.flip(w, axis=(2, 3)), (1, 0, 2, 3))
    out = lax.conv_general_dilated(
        x, w2, (1, 1), ((k - 1 - pad, k - 1 - pad), (k - 1 - pad, k - 1 - pad)),
        lhs_dilation=(stride, stride), dimension_numbers=('NCHW', 'OIHW', 'NCHW'))
    return out + b[None, :, None, None]


def _bn(x, g, b):
    m = jnp.mean(x, axis=(0, 2, 3), keepdims=True)
    v = jnp.var(x, axis=(0, 2, 3), keepdims=True)
    return (x - m) / jnp.sqrt(v + _EPS) * g[None, :, None, None] + b[None, :, None, None]


def _relu(x):
    return jnp.maximum(x, 0.0)


def _encoder(x, p):
    h = _relu(_bn(_conv(x, p['enc_c1_w'], p['enc_c1_b'], 2, 1), p['enc_bn1_g'], p['enc_bn1_b']))
    h = _relu(_bn(_conv(h, p['enc_c2_w'], p['enc_c2_b'], 2, 1), p['enc_bn2_g'], p['enc_bn2_b']))
    h = h + _relu(_bn(_conv(h, p['enc_r1_w'], p['enc_r1_b'], 1, 1), p['enc_r1bn_g'], p['enc_r1bn_b']))
    h = h + _relu(_bn(_conv(h, p['enc_r2_w'], p['enc_r2_b'], 1, 0), p['enc_r2bn_g'], p['enc_r2bn_b']))
    return _conv(h, p['enc_proj_w'], p['enc_proj_b'], 1, 0)


def _decoder(z, p):
    h = _conv(z, p['dec_proj_w'], p['dec_proj_b'], 1, 0)
    h = h + _relu(_bn(_conv(h, p['dec_r1_w'], p['dec_r1_b'], 1, 1), p['dec_r1bn_g'], p['dec_r1bn_b']))
    h = h + _relu(_bn(_conv(h, p['dec_r2_w'], p['dec_r2_b'], 1, 1), p['dec_r2bn_g'], p['dec_r2bn_b']))
    h = _relu(_bn(_convT(h, p['dec_ct1_w'], p['dec_ct1_b'], 2, 1), p['dec_ctbn_g'], p['dec_ctbn_b']))
    h = _convT(h, p['dec_ct2_w'], p['dec_ct2_b'], 2, 1)
    return jax.nn.sigmoid(h)


def kernel(x, params):
    enc = _encoder(x, params)
    b, c, hh, ww = enc.shape
    q2d = enc.reshape(-1, _D)
    emb = params['emb']
    emb_bf = emb.astype(jnp.bfloat16)
    he = (0.5 * jnp.sum(emb * emb, -1)).reshape(_K_STEPS, _K_BLK)
    closest_flat, qloss = _vq_argmin_loss(q2d, emb_bf, he)
    idx_padded = jnp.concatenate(
        [closest_flat, jnp.zeros((_PAD_TOK - _N_TOK,), jnp.int32)])
    quant = _sc_gather(emb, idx_padded)[:_N_TOK]
    quant_out = (q2d + (quant - q2d)).reshape(b, c, hh, ww)
    out = _decoder(quant_out, params)
    closest = closest_flat.reshape(b, -1)
    return out, closest, qloss
